# baseline (speedup 1.0000x reference)
import functools

import jax
import jax.numpy as jnp
from jax import lax
from jax.experimental import pallas as pl
from jax.experimental.pallas import tpu as pltpu
from jax.experimental.pallas import tpu_sc as plsc

N_NODES = 10000
D = 128
TIMESTEP = 3

ROW_BLK = 2000

NTILES = 16
CHUNK = 128
EDGES = 160000
NCHUNK = 80
EPT = NCHUNK * CHUNK
STRIDE = 9984
NPAD = 10240
SLICE = NPAD // NTILES


def _tc_body(x_ref, deg_ref, win_ref, wout_ref, wih_ref, whh_ref,
             bih_ref, bhh_ref, out_ref, degt_scr):
    i = pl.program_id(0)
    xb = x_ref[...]
    dn = (((1,), (1,)), ((), ()))

    @pl.when(i == 0)
    def _():
        degt_scr[...] = jnp.transpose(deg_ref[...])

    degb = degt_scr[pl.ds(i * ROW_BLK, ROW_BLK), :]
    bf = jnp.bfloat16
    xb16 = xb.astype(bf)
    wio16 = jnp.concatenate([win_ref[...], wout_ref[...]], axis=0).astype(bf)
    po = lax.dot_general(xb16, wio16, dn,
                         preferred_element_type=jnp.float32)
    a_in = jnp.where(degb[:, 0:1] > 0.0, po[:, :D], 0.0)
    a_out = jnp.where(degb[:, 1:2] > 0.0, po[:, D:], 0.0)
    gf = jnp.maximum(a_in, a_out)
    wih16 = wih_ref[...].astype(bf)
    whh16 = whh_ref[...].astype(bf)

    def sigmoid(v):
        return 0.5 * jnp.tanh(0.5 * v) + 0.5

    gi = lax.dot_general(gf.astype(bf), wih16, dn,
                         preferred_element_type=jnp.float32) + bih_ref[...]
    h = xb
    for _ in range(TIMESTEP):
        gh = lax.dot_general(h.astype(bf), whh16, dn,
                             preferred_element_type=jnp.float32) + bhh_ref[...]
        r = sigmoid(gi[:, :D] + gh[:, :D])
        z = sigmoid(gi[:, D:2 * D] + gh[:, D:2 * D])
        n = jnp.tanh(gi[:, 2 * D:] + r * gh[:, 2 * D:])
        h = n + z * (h - n)
    out_ref[...] = h


def _dense_stage(x, deg, We_in, We_out, w_ih, w_hh, b_ih, b_hh):
    n = x.shape[0]
    grid = n // ROW_BLK
    wspec = lambda a: pl.BlockSpec(a.shape, lambda i: (0,) * a.ndim)
    return pl.pallas_call(
        _tc_body,
        grid=(grid,),
        in_specs=[
            pl.BlockSpec((ROW_BLK, D), lambda i: (i, 0)),
            wspec(deg),
            wspec(We_in), wspec(We_out), wspec(w_ih), wspec(w_hh),
            wspec(b_ih), wspec(b_hh),
        ],
        out_specs=pl.BlockSpec((ROW_BLK, D), lambda i: (i, 0)),
        out_shape=jax.ShapeDtypeStruct((n, D), jnp.float32),
        scratch_shapes=[pltpu.VMEM((NPAD, 2), jnp.float32)],
    )(x, deg, We_in, We_out, w_ih, w_hh, b_ih, b_hh)


UNROLL = 8


def _sc_mask_body(ein_ref, eout_ref, out_ref, idx_v, ones_v, zeros_v,
                  dummy_v, shared, idx_sem, sc_sem):
    c = lax.axis_index("c")
    s = lax.axis_index("s")
    base = pl.multiple_of(s * STRIDE, CHUNK)

    def start_stage(e_ref):
        def body(j8, carry):
            for k in range(UNROLL):
                j = j8 * UNROLL + k
                pltpu.make_async_copy(
                    e_ref.at[:, pl.ds(base + j * CHUNK, CHUNK)],
                    idx_v.at[j], idx_sem).start()
            return carry
        lax.fori_loop(0, NCHUNK // UNROLL, body, 0)

    @pl.when(c == 0)
    def _():
        start_stage(ein_ref)

    @pl.when(c == 1)
    def _():
        start_stage(eout_ref)

    for i in range(CHUNK // 16):
        ones_v[pl.ds(i * 16, 16)] = jnp.full((16,), 1.0, jnp.float32)
    for i in range(SLICE // 16):
        zeros_v[pl.ds(i * 16, 16)] = jnp.zeros((16,), jnp.float32)
    pltpu.sync_copy(zeros_v, shared.at[pl.ds(s * SLICE, SLICE)])

    pltpu.make_async_copy(ein_ref.at[:, pl.ds(0, NCHUNK * CHUNK)],
                          dummy_v, idx_sem).wait()
    plsc.subcore_barrier()

    def fire(j8, carry):
        for k in range(UNROLL):
            j = j8 * UNROLL + k
            pltpu.async_copy(ones_v, shared.at[idx_v.at[j, 1]], sc_sem,
                             add=True)
        return carry

    lax.fori_loop(0, NCHUNK // UNROLL, fire, 0)
    pltpu.make_async_copy(ein_ref.at[:, pl.ds(0, NCHUNK * CHUNK // 2)],
                          dummy_v.at[:, pl.ds(0, NCHUNK * CHUNK // 2)],
                          sc_sem).wait()

    plsc.subcore_barrier()
    pltpu.sync_copy(shared.at[pl.ds(s * SLICE, SLICE)],
                    out_ref.at[c, pl.ds(s * SLICE, SLICE)])


_sc_masks = pl.kernel(
    _sc_mask_body,
    out_type=jax.ShapeDtypeStruct((2, NPAD), jnp.float32),
    mesh=plsc.VectorSubcoreMesh(core_axis_name="c", subcore_axis_name="s"),
    scratch_types=[
        pltpu.VMEM((NCHUNK, 2, CHUNK), jnp.int32),
        pltpu.VMEM((CHUNK,), jnp.float32),
        pltpu.VMEM((SLICE,), jnp.float32),
        pltpu.VMEM((2, NCHUNK * CHUNK), jnp.int32),
        pltpu.VMEM_SHARED((NPAD,), jnp.float32),
        pltpu.SemaphoreType.DMA,
        pltpu.SemaphoreType.DMA,
    ],
)


def kernel(x, We_in, We_out, w_ih, w_hh, b_ih, b_hh, edge_index_in,
           edge_index_out):
    deg = _sc_masks(edge_index_in.astype(jnp.int32),
                    edge_index_out.astype(jnp.int32))
    return _dense_stage(x, deg, We_in, We_out, w_ih, w_hh,
                        b_ih.reshape(1, 3 * D), b_hh.reshape(1, 3 * D))


# speedup vs baseline: 1.0107x; 1.0107x over previous
import functools

import jax
import jax.numpy as jnp
from jax import lax
from jax.experimental import pallas as pl
from jax.experimental.pallas import tpu as pltpu
from jax.experimental.pallas import tpu_sc as plsc

N_NODES = 10000
D = 128
TIMESTEP = 3

ROW_BLK = 2000

NTILES = 16
CHUNK = 128
EDGES = 160000
NCHUNK = 80
EPT = NCHUNK * CHUNK
STRIDE = 9984
NPAD = 10240
SLICE = NPAD // NTILES


def _tc_body(x_ref, deg_ref, win_ref, wout_ref, wih_ref, whh_ref,
             bih_ref, bhh_ref, out_ref, degt_scr):
    i = pl.program_id(0)
    xb = x_ref[...]
    dn = (((1,), (1,)), ((), ()))

    @pl.when(i == 0)
    def _():
        degt_scr[...] = jnp.transpose(deg_ref[...])

    degb = degt_scr[pl.ds(i * ROW_BLK, ROW_BLK), :]
    bf = jnp.bfloat16
    xb16 = xb.astype(bf)
    wio16 = jnp.concatenate([win_ref[...], wout_ref[...]], axis=0).astype(bf)
    po = lax.dot_general(xb16, wio16, dn,
                         preferred_element_type=jnp.float32)
    a_in = jnp.where(degb[:, 0:1] > 0.0, po[:, :D], 0.0)
    a_out = jnp.where(degb[:, 1:2] > 0.0, po[:, D:], 0.0)
    gf = jnp.maximum(a_in, a_out)
    wih16 = wih_ref[...].astype(bf)
    whh16 = whh_ref[...].astype(bf)

    def sigmoid(v):
        return 0.5 * jnp.tanh(0.5 * v) + 0.5

    gi = lax.dot_general(gf.astype(bf), wih16, dn,
                         preferred_element_type=jnp.float32) + bih_ref[...]
    h = xb
    for _ in range(TIMESTEP):
        gh = lax.dot_general(h.astype(bf), whh16, dn,
                             preferred_element_type=jnp.float32) + bhh_ref[...]
        r = sigmoid(gi[:, :D] + gh[:, :D])
        z = sigmoid(gi[:, D:2 * D] + gh[:, D:2 * D])
        n = jnp.tanh(gi[:, 2 * D:] + r * gh[:, 2 * D:])
        h = n + z * (h - n)
    out_ref[...] = h


def _dense_stage(x, deg, We_in, We_out, w_ih, w_hh, b_ih, b_hh):
    n = x.shape[0]
    grid = n // ROW_BLK
    wspec = lambda a: pl.BlockSpec(a.shape, lambda i: (0,) * a.ndim)
    return pl.pallas_call(
        _tc_body,
        grid=(grid,),
        in_specs=[
            pl.BlockSpec((ROW_BLK, D), lambda i: (i, 0)),
            wspec(deg),
            wspec(We_in), wspec(We_out), wspec(w_ih), wspec(w_hh),
            wspec(b_ih), wspec(b_hh),
        ],
        out_specs=pl.BlockSpec((ROW_BLK, D), lambda i: (i, 0)),
        out_shape=jax.ShapeDtypeStruct((n, D), jnp.float32),
        scratch_shapes=[pltpu.VMEM((NPAD, 2), jnp.float32)],
    )(x, deg, We_in, We_out, w_ih, w_hh, b_ih, b_hh)


UNROLL = 1


def _sc_mask_body(ein_ref, eout_ref, out_ref, idx_v, ones_v, zeros_v,
                  dummy_v, shared, idx_sem, sc_sem):
    c = lax.axis_index("c")
    s = lax.axis_index("s")
    base = pl.multiple_of(s * STRIDE, CHUNK)

    def start_stage(e_ref):
        def body(j8, carry):
            for k in range(UNROLL):
                j = j8 * UNROLL + k
                pltpu.make_async_copy(
                    e_ref.at[:, pl.ds(base + j * CHUNK, CHUNK)],
                    idx_v.at[j], idx_sem).start()
            return carry
        lax.fori_loop(0, NCHUNK // UNROLL, body, 0)

    @pl.when(c == 0)
    def _():
        start_stage(ein_ref)

    @pl.when(c == 1)
    def _():
        start_stage(eout_ref)

    for i in range(CHUNK // 16):
        ones_v[pl.ds(i * 16, 16)] = jnp.full((16,), 1.0, jnp.float32)
    for i in range(SLICE // 16):
        zeros_v[pl.ds(i * 16, 16)] = jnp.zeros((16,), jnp.float32)
    pltpu.sync_copy(zeros_v, shared.at[pl.ds(s * SLICE, SLICE)])

    pltpu.make_async_copy(ein_ref.at[:, pl.ds(0, NCHUNK * CHUNK)],
                          dummy_v, idx_sem).wait()
    plsc.subcore_barrier()

    def fire(j8, carry):
        for k in range(UNROLL):
            j = j8 * UNROLL + k
            pltpu.async_copy(ones_v, shared.at[idx_v.at[j, 1]], sc_sem,
                             add=True)
        return carry

    lax.fori_loop(0, NCHUNK // UNROLL, fire, 0)
    pltpu.make_async_copy(ein_ref.at[:, pl.ds(0, NCHUNK * CHUNK // 2)],
                          dummy_v.at[:, pl.ds(0, NCHUNK * CHUNK // 2)],
                          sc_sem).wait()

    plsc.subcore_barrier()
    pltpu.sync_copy(shared.at[pl.ds(s * SLICE, SLICE)],
                    out_ref.at[c, pl.ds(s * SLICE, SLICE)])


_sc_masks = pl.kernel(
    _sc_mask_body,
    out_type=jax.ShapeDtypeStruct((2, NPAD), jnp.float32),
    mesh=plsc.VectorSubcoreMesh(core_axis_name="c", subcore_axis_name="s"),
    scratch_types=[
        pltpu.VMEM((NCHUNK, 2, CHUNK), jnp.int32),
        pltpu.VMEM((CHUNK,), jnp.float32),
        pltpu.VMEM((SLICE,), jnp.float32),
        pltpu.VMEM((2, NCHUNK * CHUNK), jnp.int32),
        pltpu.VMEM_SHARED((NPAD,), jnp.float32),
        pltpu.SemaphoreType.DMA,
        pltpu.SemaphoreType.DMA,
    ],
)


def kernel(x, We_in, We_out, w_ih, w_hh, b_ih, b_hh, edge_index_in,
           edge_index_out):
    deg = _sc_masks(edge_index_in.astype(jnp.int32),
                    edge_index_out.astype(jnp.int32))
    return _dense_stage(x, deg, We_in, We_out, w_ih, w_hh,
                        b_ih.reshape(1, 3 * D), b_hh.reshape(1, 3 * D))


# speedup vs baseline: 1.0187x; 1.0079x over previous
import functools

import jax
import jax.numpy as jnp
from jax import lax
from jax.experimental import pallas as pl
from jax.experimental.pallas import tpu as pltpu
from jax.experimental.pallas import tpu_sc as plsc

N_NODES = 10000
D = 128
TIMESTEP = 3

ROW_BLK = 5000

NTILES = 16
CHUNK = 128
EDGES = 160000
NCHUNK = 80
EPT = NCHUNK * CHUNK
STRIDE = 9984
NPAD = 10240
SLICE = NPAD // NTILES


def _tc_body(x_ref, deg_ref, win_ref, wout_ref, wih_ref, whh_ref,
             bih_ref, bhh_ref, out_ref, degt_scr):
    i = pl.program_id(0)
    xb = x_ref[...]
    dn = (((1,), (1,)), ((), ()))

    @pl.when(i == 0)
    def _():
        degt_scr[...] = jnp.transpose(deg_ref[...])

    degb = degt_scr[pl.ds(i * ROW_BLK, ROW_BLK), :]
    bf = jnp.bfloat16
    xb16 = xb.astype(bf)
    wio16 = jnp.concatenate([win_ref[...], wout_ref[...]], axis=0).astype(bf)
    po = lax.dot_general(xb16, wio16, dn,
                         preferred_element_type=jnp.float32)
    a_in = jnp.where(degb[:, 0:1] > 0.0, po[:, :D], 0.0)
    a_out = jnp.where(degb[:, 1:2] > 0.0, po[:, D:], 0.0)
    gf = jnp.maximum(a_in, a_out)
    wih16 = wih_ref[...].astype(bf)
    whh16 = whh_ref[...].astype(bf)

    def sigmoid(v):
        return 0.5 * jnp.tanh(0.5 * v) + 0.5

    gi = lax.dot_general(gf.astype(bf), wih16, dn,
                         preferred_element_type=jnp.float32) + bih_ref[...]
    h = xb
    for _ in range(TIMESTEP):
        gh = lax.dot_general(h.astype(bf), whh16, dn,
                             preferred_element_type=jnp.float32) + bhh_ref[...]
        r = sigmoid(gi[:, :D] + gh[:, :D])
        z = sigmoid(gi[:, D:2 * D] + gh[:, D:2 * D])
        n = jnp.tanh(gi[:, 2 * D:] + r * gh[:, 2 * D:])
        h = n + z * (h - n)
    out_ref[...] = h


def _dense_stage(x, deg, We_in, We_out, w_ih, w_hh, b_ih, b_hh):
    n = x.shape[0]
    grid = n // ROW_BLK
    wspec = lambda a: pl.BlockSpec(a.shape, lambda i: (0,) * a.ndim)
    return pl.pallas_call(
        _tc_body,
        grid=(grid,),
        in_specs=[
            pl.BlockSpec((ROW_BLK, D), lambda i: (i, 0)),
            wspec(deg),
            wspec(We_in), wspec(We_out), wspec(w_ih), wspec(w_hh),
            wspec(b_ih), wspec(b_hh),
        ],
        out_specs=pl.BlockSpec((ROW_BLK, D), lambda i: (i, 0)),
        out_shape=jax.ShapeDtypeStruct((n, D), jnp.float32),
        scratch_shapes=[pltpu.VMEM((NPAD, 2), jnp.float32)],
    )(x, deg, We_in, We_out, w_ih, w_hh, b_ih, b_hh)


UNROLL = 1


def _sc_mask_body(ein_ref, eout_ref, out_ref, idx_v, ones_v, zeros_v,
                  dummy_v, shared, idx_sem, sc_sem):
    c = lax.axis_index("c")
    s = lax.axis_index("s")
    base = pl.multiple_of(s * STRIDE, CHUNK)

    def start_stage(e_ref):
        def body(j8, carry):
            for k in range(UNROLL):
                j = j8 * UNROLL + k
                pltpu.make_async_copy(
                    e_ref.at[:, pl.ds(base + j * CHUNK, CHUNK)],
                    idx_v.at[j], idx_sem).start()
            return carry
        lax.fori_loop(0, NCHUNK // UNROLL, body, 0)

    @pl.when(c == 0)
    def _():
        start_stage(ein_ref)

    @pl.when(c == 1)
    def _():
        start_stage(eout_ref)

    for i in range(CHUNK // 16):
        ones_v[pl.ds(i * 16, 16)] = jnp.full((16,), 1.0, jnp.float32)
    for i in range(SLICE // 16):
        zeros_v[pl.ds(i * 16, 16)] = jnp.zeros((16,), jnp.float32)
    pltpu.sync_copy(zeros_v, shared.at[pl.ds(s * SLICE, SLICE)])

    pltpu.make_async_copy(ein_ref.at[:, pl.ds(0, NCHUNK * CHUNK)],
                          dummy_v, idx_sem).wait()
    plsc.subcore_barrier()

    def fire(j8, carry):
        for k in range(UNROLL):
            j = j8 * UNROLL + k
            pltpu.async_copy(ones_v, shared.at[idx_v.at[j, 1]], sc_sem,
                             add=True)
        return carry

    lax.fori_loop(0, NCHUNK // UNROLL, fire, 0)
    pltpu.make_async_copy(ein_ref.at[:, pl.ds(0, NCHUNK * CHUNK // 2)],
                          dummy_v.at[:, pl.ds(0, NCHUNK * CHUNK // 2)],
                          sc_sem).wait()

    plsc.subcore_barrier()
    pltpu.sync_copy(shared.at[pl.ds(s * SLICE, SLICE)],
                    out_ref.at[c, pl.ds(s * SLICE, SLICE)])


_sc_masks = pl.kernel(
    _sc_mask_body,
    out_type=jax.ShapeDtypeStruct((2, NPAD), jnp.float32),
    mesh=plsc.VectorSubcoreMesh(core_axis_name="c", subcore_axis_name="s"),
    scratch_types=[
        pltpu.VMEM((NCHUNK, 2, CHUNK), jnp.int32),
        pltpu.VMEM((CHUNK,), jnp.float32),
        pltpu.VMEM((SLICE,), jnp.float32),
        pltpu.VMEM((2, NCHUNK * CHUNK), jnp.int32),
        pltpu.VMEM_SHARED((NPAD,), jnp.float32),
        pltpu.SemaphoreType.DMA,
        pltpu.SemaphoreType.DMA,
    ],
)


def kernel(x, We_in, We_out, w_ih, w_hh, b_ih, b_hh, edge_index_in,
           edge_index_out):
    deg = _sc_masks(edge_index_in.astype(jnp.int32),
                    edge_index_out.astype(jnp.int32))
    return _dense_stage(x, deg, We_in, We_out, w_ih, w_hh,
                        b_ih.reshape(1, 3 * D), b_hh.reshape(1, 3 * D))
